# baseline (speedup 1.0000x reference)
import functools

import jax
import jax.numpy as jnp
from jax import lax
from jax.experimental import pallas as pl
from jax.experimental.pallas import tpu as pltpu
from jax.experimental.pallas import tpu_sc as plsc

_NUM_WORKERS = 32
_L = 16
_NB = 8


@jax.jit
def _gather_sc(table_t, sorted_r, order, lord, dist):
    d, _ = table_t.shape
    b = sorted_r.shape[0]
    b_per_w = b // _NUM_WORKERS
    n_pairs = b_per_w // _L // 2
    nk = d // _L
    mesh = plsc.VectorSubcoreMesh(core_axis_name="c", subcore_axis_name="s")

    @functools.partial(
        pl.kernel,
        mesh=mesh,
        out_type=jax.ShapeDtypeStruct((b, d), jnp.float32),
        scratch_types=[
            pltpu.VMEM((b_per_w,), jnp.int32),
            pltpu.VMEM((b_per_w,), jnp.int32),
            pltpu.VMEM((b_per_w,), jnp.int32),
            pltpu.VMEM((b_per_w,), jnp.int32),
            pltpu.VMEM((_NB, d, 128), jnp.float32),
            pltpu.VMEM((_L, d), jnp.float32),
            pltpu.VMEM((_L, d), jnp.float32),
            [pltpu.SemaphoreType.DMA] * _NB,
            pltpu.SemaphoreType.DMA,
            pltpu.SemaphoreType.DMA,
        ],
        compiler_params=pltpu.CompilerParams(needs_layout_passes=False),
    )
    def k(table_hbm, srt_hbm, ord_hbm, lord_hbm, dist_hbm, out_hbm,
          srt_v, ord_v, lord_v, dist_v, blk_v, row_a, row_b, bsems, sem_a, sem_b):
        wid = lax.axis_index("s") * 2 + lax.axis_index("c")
        base = wid * b_per_w
        pltpu.sync_copy(srt_hbm.at[pl.ds(base, b_per_w)], srt_v)
        pltpu.sync_copy(ord_hbm.at[pl.ds(base, b_per_w)], ord_v)
        pltpu.sync_copy(lord_hbm.at[pl.ds(base, b_per_w)], lord_v)
        pltpu.sync_copy(dist_hbm.at[pl.ds(base, b_per_w)], dist_v)
        lanes = [jnp.arange(_L, dtype=jnp.int32) + _L * kk for kk in range(nk)]
        iota = jnp.arange(_L, dtype=jnp.int32)

        def fetch(blkid, q):
            pltpu.async_copy(
                table_hbm.at[:, pl.ds(pl.multiple_of(blkid * 128, 128), 128)],
                blk_v.at[q],
                bsems[q],
            )

        def dist_at(n):
            nb16 = pl.multiple_of((n >> 4) << 4, _L)
            dvec = dist_v[pl.ds(nb16, _L)]
            return jnp.sum(jnp.where(iota == (n & (_L - 1)), dvec, 0))

        dvec0 = dist_v[pl.ds(0, _L)]
        for q in range(_NB - 1):
            fetch(dvec0[q], q)

        def half(g, cur, row_v, osem):
            rvec = srt_v[pl.ds(g * _L, _L)]
            pvec = ord_v[pl.ds(g * _L, _L)]
            lvec = lord_v[pl.ds(g * _L, _L)]
            for j in range(_L):
                r = rvec[j]
                p = pvec[j]
                lo = lvec[j]
                c = r & 127

                @pl.when(lo != cur)
                def _():
                    nxt = jnp.minimum(lo + _NB - 1, b_per_w - 1)
                    blk_nxt = dist_at(nxt)
                    for qq in range(_NB):
                        @pl.when((lo & (_NB - 1)) == qq)
                        def _():
                            pltpu.make_async_copy(
                                table_hbm.at[:, pl.ds(0, 128)],
                                blk_v.at[qq],
                                bsems[qq],
                            ).wait()
                            fetch(blk_nxt, (qq + _NB - 1) % _NB)

                cur = jnp.where(lo != cur, lo, cur)
                cvec = jnp.full((_L,), c, dtype=jnp.int32)
                qvec = jnp.full((_L,), lo & (_NB - 1), dtype=jnp.int32)
                for kk in range(nk):
                    row_v[j, pl.ds(kk * _L, _L)] = plsc.load_gather(
                        blk_v, [qvec, lanes[kk], cvec]
                    )
                pltpu.async_copy(
                    row_v.at[pl.ds(j, 1), :], out_hbm.at[pl.ds(p, 1), :], osem
                )
            return cur

        def pair(h, cur):
            @pl.when(h > 0)
            def _():
                pltpu.make_async_copy(out_hbm.at[pl.ds(0, _L), :], row_a, sem_a).wait()
            cur = half(2 * h, cur, row_a, sem_a)

            @pl.when(h > 0)
            def _():
                pltpu.make_async_copy(out_hbm.at[pl.ds(0, _L), :], row_b, sem_b).wait()
            cur = half(2 * h + 1, cur, row_b, sem_b)
            return cur

        cur = lax.fori_loop(0, n_pairs, pair, jnp.int32(-1))
        pltpu.make_async_copy(out_hbm.at[pl.ds(0, _L), :], row_a, sem_a).wait()
        pltpu.make_async_copy(out_hbm.at[pl.ds(0, _L), :], row_b, sem_b).wait()
        for qq in range(_NB):
            @pl.when((cur & (_NB - 1)) != qq)
            def _():
                pltpu.make_async_copy(
                    table_hbm.at[:, pl.ds(0, 128)], blk_v.at[qq], bsems[qq]
                ).wait()

    return k(table_t, sorted_r, order, lord, dist)


def kernel(tensor, index):
    idx = index.reshape(-1).astype(jnp.int32)
    n = idx.shape[0]
    bpw = n // _NUM_WORKERS
    pos = jnp.arange(n, dtype=jnp.int32)
    sorted_r, order = lax.sort((idx, pos), num_keys=1)
    blk = sorted_r >> 7
    seg_first = (pos % bpw) == 0
    newb = jnp.concatenate([jnp.ones((1,), bool), blk[1:] != blk[:-1]]) | seg_first
    nb32 = newb.astype(jnp.int32).reshape(_NUM_WORKERS, bpw)
    lord = (jnp.cumsum(nb32, axis=1) - 1).reshape(-1).astype(jnp.int32)
    big = jnp.int32(1 << 20)
    dist = jnp.sort(
        jnp.where(newb, blk, big).reshape(_NUM_WORKERS, bpw), axis=1
    ).reshape(-1)
    nblk = (tensor.shape[0] + 127) // 128
    dist = jnp.minimum(dist, nblk - 1).astype(jnp.int32)
    return _gather_sc(tensor.T, sorted_r, order, lord, dist)


# speedup vs baseline: 1.0167x; 1.0167x over previous
import functools

import jax
import jax.numpy as jnp
from jax import lax
from jax.experimental import pallas as pl
from jax.experimental.pallas import tpu as pltpu
from jax.experimental.pallas import tpu_sc as plsc

_NUM_WORKERS = 32
_L = 16
_NB = 4


@jax.jit
def _gather_sc(table_t, sorted_r, order, lord, dist):
    d, _ = table_t.shape
    b = sorted_r.shape[0]
    b_per_w = b // _NUM_WORKERS
    n_pairs = b_per_w // _L // 2
    nk = d // _L
    mesh = plsc.VectorSubcoreMesh(core_axis_name="c", subcore_axis_name="s")

    @functools.partial(
        pl.kernel,
        mesh=mesh,
        out_type=jax.ShapeDtypeStruct((b, d), jnp.float32),
        scratch_types=[
            pltpu.VMEM((b_per_w,), jnp.int32),
            pltpu.VMEM((b_per_w,), jnp.int32),
            pltpu.VMEM((b_per_w,), jnp.int32),
            pltpu.VMEM((b_per_w,), jnp.int32),
            pltpu.VMEM((_NB, d, 128), jnp.float32),
            pltpu.VMEM((_L, d), jnp.float32),
            pltpu.VMEM((_L, d), jnp.float32),
            [pltpu.SemaphoreType.DMA] * _NB,
            pltpu.SemaphoreType.DMA,
            pltpu.SemaphoreType.DMA,
        ],
        compiler_params=pltpu.CompilerParams(needs_layout_passes=False),
    )
    def k(table_hbm, srt_hbm, ord_hbm, lord_hbm, dist_hbm, out_hbm,
          srt_v, ord_v, lord_v, dist_v, blk_v, row_a, row_b, bsems, sem_a, sem_b):
        wid = lax.axis_index("s") * 2 + lax.axis_index("c")
        base = wid * b_per_w
        pltpu.sync_copy(srt_hbm.at[pl.ds(base, b_per_w)], srt_v)
        pltpu.sync_copy(ord_hbm.at[pl.ds(base, b_per_w)], ord_v)
        pltpu.sync_copy(lord_hbm.at[pl.ds(base, b_per_w)], lord_v)
        pltpu.sync_copy(dist_hbm.at[pl.ds(base, b_per_w)], dist_v)
        lanes = [jnp.arange(_L, dtype=jnp.int32) + _L * kk for kk in range(nk)]
        iota = jnp.arange(_L, dtype=jnp.int32)

        def fetch(blkid, q):
            pltpu.async_copy(
                table_hbm.at[:, pl.ds(pl.multiple_of(blkid * 128, 128), 128)],
                blk_v.at[q],
                bsems[q],
            )

        def dist_at(n):
            nb16 = pl.multiple_of((n >> 4) << 4, _L)
            dvec = dist_v[pl.ds(nb16, _L)]
            return jnp.sum(jnp.where(iota == (n & (_L - 1)), dvec, 0))

        dvec0 = dist_v[pl.ds(0, _L)]
        for q in range(_NB - 1):
            fetch(dvec0[q], q)

        def half(g, cur, row_v, osem):
            rvec = srt_v[pl.ds(g * _L, _L)]
            pvec = ord_v[pl.ds(g * _L, _L)]
            lvec = lord_v[pl.ds(g * _L, _L)]
            for j in range(_L):
                r = rvec[j]
                p = pvec[j]
                lo = lvec[j]
                c = r & 127

                @pl.when(lo != cur)
                def _():
                    nxt = jnp.minimum(lo + _NB - 1, b_per_w - 1)
                    blk_nxt = dist_at(nxt)
                    for qq in range(_NB):
                        @pl.when((lo & (_NB - 1)) == qq)
                        def _():
                            pltpu.make_async_copy(
                                table_hbm.at[:, pl.ds(0, 128)],
                                blk_v.at[qq],
                                bsems[qq],
                            ).wait()
                            fetch(blk_nxt, (qq + _NB - 1) % _NB)

                cur = jnp.where(lo != cur, lo, cur)
                cvec = jnp.full((_L,), c, dtype=jnp.int32)
                qvec = jnp.full((_L,), lo & (_NB - 1), dtype=jnp.int32)
                for kk in range(nk):
                    row_v[j, pl.ds(kk * _L, _L)] = plsc.load_gather(
                        blk_v, [qvec, lanes[kk], cvec]
                    )
                pltpu.async_copy(
                    row_v.at[pl.ds(j, 1), :], out_hbm.at[pl.ds(p, 1), :], osem
                )
            return cur

        def pair(h, cur):
            @pl.when(h > 0)
            def _():
                pltpu.make_async_copy(out_hbm.at[pl.ds(0, _L), :], row_a, sem_a).wait()
            cur = half(2 * h, cur, row_a, sem_a)

            @pl.when(h > 0)
            def _():
                pltpu.make_async_copy(out_hbm.at[pl.ds(0, _L), :], row_b, sem_b).wait()
            cur = half(2 * h + 1, cur, row_b, sem_b)
            return cur

        cur = lax.fori_loop(0, n_pairs, pair, jnp.int32(-1))
        pltpu.make_async_copy(out_hbm.at[pl.ds(0, _L), :], row_a, sem_a).wait()
        pltpu.make_async_copy(out_hbm.at[pl.ds(0, _L), :], row_b, sem_b).wait()
        for qq in range(_NB):
            @pl.when((cur & (_NB - 1)) != qq)
            def _():
                pltpu.make_async_copy(
                    table_hbm.at[:, pl.ds(0, 128)], blk_v.at[qq], bsems[qq]
                ).wait()

    return k(table_t, sorted_r, order, lord, dist)


def kernel(tensor, index):
    idx = index.reshape(-1).astype(jnp.int32)
    n = idx.shape[0]
    bpw = n // _NUM_WORKERS
    pos = jnp.arange(n, dtype=jnp.int32)
    sorted_r, order = lax.sort((idx, pos), num_keys=1)
    blk = sorted_r >> 7
    seg_first = (pos % bpw) == 0
    newb = jnp.concatenate([jnp.ones((1,), bool), blk[1:] != blk[:-1]]) | seg_first
    nb32 = newb.astype(jnp.int32).reshape(_NUM_WORKERS, bpw)
    lord = (jnp.cumsum(nb32, axis=1) - 1).reshape(-1).astype(jnp.int32)
    big = jnp.int32(1 << 20)
    dist = jnp.sort(
        jnp.where(newb, blk, big).reshape(_NUM_WORKERS, bpw), axis=1
    ).reshape(-1)
    nblk = (tensor.shape[0] + 127) // 128
    dist = jnp.minimum(dist, nblk - 1).astype(jnp.int32)
    return _gather_sc(tensor.T, sorted_r, order, lord, dist)


# speedup vs baseline: 1.0679x; 1.0504x over previous
import functools

import jax
import jax.numpy as jnp
from jax import lax
from jax.experimental import pallas as pl
from jax.experimental.pallas import tpu as pltpu
from jax.experimental.pallas import tpu_sc as plsc

_NUM_WORKERS = 32
_L = 16
_NB = 4


@jax.jit
def _gather_sc(table_t, sorted_r, order, dist):
    d, _ = table_t.shape
    b = sorted_r.shape[0]
    b_per_w = b // _NUM_WORKERS
    n_groups = b_per_w // _L
    nk = d // _L
    mesh = plsc.VectorSubcoreMesh(core_axis_name="c", subcore_axis_name="s")

    @functools.partial(
        pl.kernel,
        mesh=mesh,
        out_type=jax.ShapeDtypeStruct((b, d), jnp.float32),
        scratch_types=[
            pltpu.VMEM((b_per_w,), jnp.int32),
            pltpu.VMEM((b_per_w,), jnp.int32),
            pltpu.VMEM((b_per_w,), jnp.int32),
            pltpu.VMEM((_NB, d, 128), jnp.float32),
            pltpu.VMEM((_L, d), jnp.float32),
            [pltpu.SemaphoreType.DMA] * _NB,
            pltpu.SemaphoreType.DMA,
        ],
        compiler_params=pltpu.CompilerParams(needs_layout_passes=False),
    )
    def k(table_hbm, srt_hbm, ord_hbm, dist_hbm, out_hbm,
          srt_v, ord_v, dist_v, blk_v, row_v, bsems, osem):
        wid = lax.axis_index("s") * 2 + lax.axis_index("c")
        base = wid * b_per_w
        pltpu.async_copy(srt_hbm.at[pl.ds(base, b_per_w)], srt_v, osem)
        pltpu.async_copy(ord_hbm.at[pl.ds(base, b_per_w)], ord_v, osem)
        pltpu.async_copy(dist_hbm.at[pl.ds(base, b_per_w)], dist_v, osem)
        pltpu.make_async_copy(srt_hbm.at[pl.ds(0, b_per_w)], srt_v, osem).wait()
        pltpu.make_async_copy(srt_hbm.at[pl.ds(0, b_per_w)], ord_v, osem).wait()
        pltpu.make_async_copy(srt_hbm.at[pl.ds(0, b_per_w)], dist_v, osem).wait()
        lanes = [jnp.arange(_L, dtype=jnp.int32) + _L * kk for kk in range(nk)]
        iota = jnp.arange(_L, dtype=jnp.int32)

        def fetch(blkid, q):
            pltpu.async_copy(
                table_hbm.at[:, pl.ds(pl.multiple_of(blkid * 128, 128), 128)],
                blk_v.at[q],
                bsems[q],
            )

        def dist_at(n):
            nb16 = pl.multiple_of((n >> 4) << 4, _L)
            dvec = dist_v[pl.ds(nb16, _L)]
            return jnp.sum(jnp.where(iota == (n & (_L - 1)), dvec, 0))

        dvec0 = dist_v[pl.ds(0, _L)]
        for q in range(_NB - 1):
            fetch(dvec0[q], q)

        def group(g, carry):
            cur, rd = carry
            rvec = srt_v[pl.ds(g * _L, _L)]
            pvec = ord_v[pl.ds(g * _L, _L)]
            for j in range(_L):
                r = rvec[j]
                p = pvec[j]
                blk = r >> 7
                c = r & 127
                adv = blk != cur

                @pl.when(adv)
                def _():
                    nxt = jnp.minimum(rd + _NB - 1, b_per_w - 1)
                    blk_nxt = dist_at(nxt)
                    for qq in range(_NB):
                        @pl.when((rd & (_NB - 1)) == qq)
                        def _():
                            pltpu.make_async_copy(
                                table_hbm.at[:, pl.ds(0, 128)],
                                blk_v.at[qq],
                                bsems[qq],
                            ).wait()
                            fetch(blk_nxt, (qq + _NB - 1) % _NB)

                cur = jnp.where(adv, blk, cur)
                rd = jnp.where(adv, rd + 1, rd)
                cvec = jnp.full((_L,), c, dtype=jnp.int32)
                qvec = jnp.full((_L,), (rd - 1) & (_NB - 1), dtype=jnp.int32)
                for kk in range(nk):
                    row_v[j, pl.ds(kk * _L, _L)] = plsc.load_gather(
                        blk_v, [qvec, lanes[kk], cvec]
                    )
                pltpu.async_copy(
                    row_v.at[pl.ds(j, 1), :], out_hbm.at[pl.ds(p, 1), :], osem
                )
            pltpu.make_async_copy(out_hbm.at[pl.ds(0, _L), :], row_v, osem).wait()
            return cur, rd

        _, rd = lax.fori_loop(
            0, n_groups, group, (jnp.int32(-1), jnp.int32(0))
        )
        for qq in range(_NB):
            @pl.when(((rd - 1) & (_NB - 1)) != qq)
            def _():
                pltpu.make_async_copy(
                    table_hbm.at[:, pl.ds(0, 128)], blk_v.at[qq], bsems[qq]
                ).wait()

    return k(table_t, sorted_r, order, dist)


def kernel(tensor, index):
    idx = index.reshape(-1).astype(jnp.int32)
    n = idx.shape[0]
    bpw = n // _NUM_WORKERS
    pos = jnp.arange(n, dtype=jnp.int32)
    sorted_r, order = lax.sort((idx, pos), num_keys=1)
    blk = sorted_r >> 7
    seg_first = (pos % bpw) == 0
    newb = jnp.concatenate([jnp.ones((1,), bool), blk[1:] != blk[:-1]]) | seg_first
    big = jnp.int32(1 << 20)
    dist = jnp.sort(
        jnp.where(newb, blk, big).reshape(_NUM_WORKERS, bpw), axis=1
    ).reshape(-1)
    nblk = (tensor.shape[0] + 127) // 128
    dist = jnp.minimum(dist, nblk - 1).astype(jnp.int32)
    return _gather_sc(tensor.T, sorted_r, order, dist)
